# Initial kernel scaffold; baseline (speedup 1.0000x reference)
import functools

import jax
import jax.numpy as jnp
from jax import lax
from jax.experimental import pallas as pl
from jax.experimental.pallas import tpu as pltpu
from jax.experimental.pallas import tpu_sc as plsc

ROW = 1000
NUM_WORKERS = 32
TOTAL = 1024 * 50
PER_WORKER = TOTAL // NUM_WORKERS
CHUNK = 64
NCHUNK = PER_WORKER // CHUNK

_MESH = plsc.VectorSubcoreMesh(core_axis_name="c", subcore_axis_name="s")


@functools.partial(
    pl.kernel,
    mesh=_MESH,
    out_type=jax.ShapeDtypeStruct((TOTAL, ROW), jnp.float32),
    scratch_types=[
        pltpu.VMEM((NCHUNK, CHUNK), jnp.int32),
        pltpu.VMEM((CHUNK, ROW), jnp.float32),
        pltpu.SemaphoreType.DMA,
    ],
)
def _gather(idx_hbm, table_hbm, out_hbm, idx_v, rows_v, sem):
    wid = lax.axis_index("s") * 2 + lax.axis_index("c")
    pltpu.sync_copy(idx_hbm.at[wid], idx_v)

    def chunk_body(j, carry):
        pltpu.async_copy(table_hbm.at[idx_v.at[j]], rows_v, sem).wait()
        row0 = wid * PER_WORKER + j * CHUNK
        pltpu.sync_copy(rows_v, out_hbm.at[pl.ds(row0, CHUNK)])
        return carry

    lax.fori_loop(0, NCHUNK, chunk_body, 0)


def kernel(x, logits_table):
    idx = x.reshape(NUM_WORKERS, NCHUNK, CHUNK).astype(jnp.int32)
    out = _gather(idx, logits_table)
    return out.reshape(x.shape[0], x.shape[1], ROW)


# speedup vs baseline: 1.0140x; 1.0140x over previous
import functools

import jax
import jax.numpy as jnp
from jax import lax
from jax.experimental import pallas as pl
from jax.experimental.pallas import tpu as pltpu
from jax.experimental.pallas import tpu_sc as plsc

ROW = 1000
NUM_WORKERS = 32
TOTAL = 1024 * 50
PER_WORKER = TOTAL // NUM_WORKERS
CHUNK = 64
NCHUNK = PER_WORKER // CHUNK

_MESH = plsc.VectorSubcoreMesh(core_axis_name="c", subcore_axis_name="s")


@functools.partial(
    pl.kernel,
    mesh=_MESH,
    compiler_params=pltpu.CompilerParams(use_tc_tiling_on_sc=False),
    out_type=jax.ShapeDtypeStruct((TOTAL, ROW), jnp.float32),
    scratch_types=[
        pltpu.VMEM((NCHUNK, CHUNK), jnp.int32),
        pltpu.VMEM((CHUNK, ROW), jnp.float32),
        pltpu.SemaphoreType.DMA,
    ],
)
def _gather(idx_hbm, table_hbm, out_hbm, idx_v, rows_v, sem):
    wid = lax.axis_index("s") * 2 + lax.axis_index("c")
    pltpu.sync_copy(idx_hbm.at[wid], idx_v)

    def chunk_body(j, carry):
        pltpu.async_copy(table_hbm.at[idx_v.at[j]], rows_v, sem).wait()
        row0 = wid * PER_WORKER + j * CHUNK
        pltpu.sync_copy(rows_v, out_hbm.at[pl.ds(row0, CHUNK)])
        return carry

    lax.fori_loop(0, NCHUNK, chunk_body, 0)


def kernel(x, logits_table):
    idx = x.reshape(NUM_WORKERS, NCHUNK, CHUNK).astype(jnp.int32)
    out = _gather(idx, logits_table)
    return out.reshape(x.shape[0], x.shape[1], ROW)


# speedup vs baseline: 1.0202x; 1.0062x over previous
import functools

import jax
import jax.numpy as jnp
from jax import lax
from jax.experimental import pallas as pl
from jax.experimental.pallas import tpu as pltpu
from jax.experimental.pallas import tpu_sc as plsc

ROW = 1000
NUM_WORKERS = 32
TOTAL = 1024 * 50
PER_WORKER = TOTAL // NUM_WORKERS
CHUNK = 40
NCHUNK = PER_WORKER // CHUNK
PAIRS = NCHUNK // 2

_MESH = plsc.VectorSubcoreMesh(core_axis_name="c", subcore_axis_name="s")


@functools.partial(
    pl.kernel,
    mesh=_MESH,
    compiler_params=pltpu.CompilerParams(use_tc_tiling_on_sc=False),
    out_type=jax.ShapeDtypeStruct((TOTAL, ROW), jnp.float32),
    scratch_types=[
        pltpu.VMEM((NCHUNK, CHUNK), jnp.int32),
        pltpu.VMEM((CHUNK, ROW), jnp.float32),
        pltpu.VMEM((CHUNK, ROW), jnp.float32),
        pltpu.SemaphoreType.DMA,
        pltpu.SemaphoreType.DMA,
        pltpu.SemaphoreType.DMA,
        pltpu.SemaphoreType.DMA,
    ],
)
def _gather(idx_hbm, table_hbm, out_hbm, idx_v, b0, b1, sg0, sg1, ss0, ss1):
    wid = lax.axis_index("s") * 2 + lax.axis_index("c")
    base = wid * PER_WORKER
    pltpu.sync_copy(idx_hbm.at[wid], idx_v)

    def out_at(j):
        return out_hbm.at[pl.ds(base + j * CHUNK, CHUNK)]

    def g_start(j, buf, sem):
        return pltpu.async_copy(table_hbm.at[idx_v.at[j]], buf, sem)

    def g_wait(j, buf, sem):
        pltpu.make_async_copy(table_hbm.at[idx_v.at[j]], buf, sem).wait()

    def s_start(j, buf, sem):
        return pltpu.async_copy(buf, out_at(j), sem)

    def s_wait(j, buf, sem):
        pltpu.make_async_copy(buf, out_at(j), sem).wait()

    d = g_start(0, b0, sg0)
    d.wait()
    d0 = s_start(0, b0, ss0)
    g_start(1, b1, sg1)
    d0.wait()
    g_start(2, b0, sg0)
    g_wait(1, b1, sg1)
    s_start(1, b1, ss1)

    def body(s, carry):
        j0 = 2 * s
        j1 = j0 + 1
        g_wait(j0, b0, sg0)
        dsc = s_start(j0, b0, ss0)
        s_wait(j1 - 2, b1, ss1)
        dg = g_start(j1, b1, sg1)
        dsc.wait()
        g_start(j0 + 2, b0, sg0)
        dg.wait()
        s_start(j1, b1, ss1)
        return carry

    lax.fori_loop(1, PAIRS - 1, body, 0)

    jA = NCHUNK - 2
    jB = NCHUNK - 1
    g_wait(jA, b0, sg0)
    dA = s_start(jA, b0, ss0)
    s_wait(jA - 1, b1, ss1)
    dB = g_start(jB, b1, sg1)
    dB.wait()
    dC = s_start(jB, b1, ss1)
    dA.wait()
    dC.wait()


def kernel(x, logits_table):
    idx = x.reshape(NUM_WORKERS, NCHUNK, CHUNK).astype(jnp.int32)
    out = _gather(idx, logits_table)
    return out.reshape(x.shape[0], x.shape[1], ROW)


# speedup vs baseline: 1.0256x; 1.0053x over previous
import functools

import jax
import jax.numpy as jnp
from jax import lax
from jax.experimental import pallas as pl
from jax.experimental.pallas import tpu as pltpu
from jax.experimental.pallas import tpu_sc as plsc

B, T = 1024, 50
ROW = 1000
NUM_WORKERS = 32
PER_WORKER = B // NUM_WORKERS
PAIRS = PER_WORKER // 2

_MESH = plsc.VectorSubcoreMesh(core_axis_name="c", subcore_axis_name="s")


@functools.partial(
    pl.kernel,
    mesh=_MESH,
    compiler_params=pltpu.CompilerParams(use_tc_tiling_on_sc=False),
    out_type=jax.ShapeDtypeStruct((B, T, ROW), jnp.float32),
    scratch_types=[
        pltpu.VMEM((PER_WORKER, T), jnp.int32),
        pltpu.VMEM((T, ROW), jnp.float32),
        pltpu.VMEM((T, ROW), jnp.float32),
        pltpu.SemaphoreType.DMA,
        pltpu.SemaphoreType.DMA,
        pltpu.SemaphoreType.DMA,
        pltpu.SemaphoreType.DMA,
    ],
)
def _gather(idx_hbm, table_hbm, out_hbm, idx_v, b0, b1, sg0, sg1, ss0, ss1):
    wid = lax.axis_index("s") * 2 + lax.axis_index("c")
    base = wid * PER_WORKER
    pltpu.sync_copy(idx_hbm.at[pl.ds(base, PER_WORKER)], idx_v)

    def g_start(j, buf, sem):
        return pltpu.async_copy(table_hbm.at[idx_v.at[j]], buf, sem)

    def g_wait(j, buf, sem):
        pltpu.make_async_copy(table_hbm.at[idx_v.at[j]], buf, sem).wait()

    def s_start(j, buf, sem):
        return pltpu.async_copy(buf, out_hbm.at[base + j], sem)

    def s_wait(j, buf, sem):
        pltpu.make_async_copy(buf, out_hbm.at[base + j], sem).wait()

    d = g_start(0, b0, sg0)
    d.wait()
    d0 = s_start(0, b0, ss0)
    g_start(1, b1, sg1)
    d0.wait()
    g_start(2, b0, sg0)
    g_wait(1, b1, sg1)
    s_start(1, b1, ss1)

    def body(s, carry):
        j0 = 2 * s
        j1 = j0 + 1
        g_wait(j0, b0, sg0)
        dsc = s_start(j0, b0, ss0)
        s_wait(j1 - 2, b1, ss1)
        dg = g_start(j1, b1, sg1)
        dsc.wait()
        g_start(j0 + 2, b0, sg0)
        dg.wait()
        s_start(j1, b1, ss1)
        return carry

    lax.fori_loop(1, PAIRS - 1, body, 0)

    jA = PER_WORKER - 2
    jB = PER_WORKER - 1
    g_wait(jA, b0, sg0)
    dA = s_start(jA, b0, ss0)
    s_wait(jA - 1, b1, ss1)
    dB = g_start(jB, b1, sg1)
    dB.wait()
    dC = s_start(jB, b1, ss1)
    dA.wait()
    dC.wait()


def kernel(x, logits_table):
    return _gather(x.astype(jnp.int32), logits_table)
